# Initial kernel scaffold; baseline (speedup 1.0000x reference)
import jax
import jax.numpy as jnp
from jax.experimental import pallas as pl


def kernel(flow_traffic, flow_packets, flow_length, link_capacity, buffer_type, link_to_path, path_to_link_path, path_to_link_pos, queue_to_link, flow_packet_size_wt_0, flow_ipg_wt_0, emb_W, emb_U, emb_b, pu_W, pu_U, pu_b, qu_W, qu_U, qu_b, lu_W, lu_U, lu_b, qe_W1, qe_b1, qe_W2, qe_b2, le_W1, le_b1, le_W2, le_b2, ro_W1, ro_b1, ro_W2, ro_b2, ro_W3, ro_b3):
    raise NotImplementedError("write your pallas kernel here")


# speedup vs baseline: 2.0051x; 2.0051x over previous
import functools

import jax
import jax.numpy as jnp
from jax.experimental import pallas as pl
from jax.experimental.pallas import tpu as pltpu
from jax.experimental.pallas import tpu_sc as plsc

H = 32
ITERS = 8
F = 10000
FP = 10240
L = 2000
P = 8
PL = 32
QL = 4
T = 16
BLK = 1024

_GW = 128
_NA = P * FP
_NB = 65536
_NC = 8192


def _sc_gather(table, idx_flat, width):
    n = idx_flat.shape[0]
    mesh = plsc.VectorSubcoreMesh(core_axis_name="core", subcore_axis_name="subcore")

    @functools.partial(
        pl.kernel,
        out_type=jax.ShapeDtypeStruct((n, width), table.dtype),
        mesh=mesh,
        compiler_params=pltpu.CompilerParams(use_tc_tiling_on_sc=False),
    )
    def gk(tab_hbm, idx_hbm, out_hbm):
        def body(i_vmem, o_vmem):
            pltpu.sync_copy(tab_hbm.at[i_vmem.at[0]], o_vmem)

        pltpu.emit_pipeline(
            body,
            grid=(n // _GW,),
            in_specs=[pl.BlockSpec((1, _GW), lambda i: (0, i))],
            out_specs=[pl.BlockSpec((_GW, width), lambda i: (i, 0))],
            core_axis_name=("core", "subcore"),
            dimension_semantics=(pltpu.PARALLEL,),
        )(idx_hbm, out_hbm)

    return gk(table, idx_flat.reshape(1, n))


def _gru_update(h, xwb, hu):
    z = jax.nn.sigmoid(xwb[:, :H] + hu[:, :H])
    r = jax.nn.sigmoid(xwb[:, H:2 * H] + hu[:, H:2 * H])
    n = jnp.tanh(xwb[:, 2 * H:] + r * hu[:, 2 * H:])
    return z * h + (1.0 - z) * n


def _dot(a, b):
    return jnp.dot(a, b, preferred_element_type=jnp.float32)


def _emb_body(fps_ref, ipg_ref, w_ref, u_ref, b_ref, out_ref):
    b = b_ref[...]
    w0 = w_ref[0:1, :]
    w1 = w_ref[1:2, :]
    u = u_ref[...]
    h = jnp.zeros((BLK, H), jnp.float32)
    for t in range(T):
        xwb = fps_ref[:, t:t + 1] * w0 + ipg_ref[:, t:t + 1] * w1 + b
        hu = _dot(h, u)
        h = _gru_update(h, xwb, hu)
    out_ref[...] = h


def _path_body(xcat_ref, h0_ref, w_ref, u_ref, b_ref, pss_ref, ht_ref):
    w = w_ref[...]
    u = u_ref[...]
    b = b_ref[...]
    h = h0_ref[...]
    pss_ref[0] = h
    for p in range(P):
        x = xcat_ref[p]
        xwb = _dot(x, w) + b
        hu = _dot(h, u)
        h = _gru_update(h, xwb, hu)
        pss_ref[p + 1] = h
    ht_ref[...] = h


def _queue_body(gb_ref, qs_ref, w_ref, u_ref, b_ref, out_ref):
    acc = gb_ref[pl.ds(0, L), :]
    for j in range(1, PL):
        acc = acc + gb_ref[pl.ds(j * L, L), :]
    h = qs_ref[...]
    xwb = _dot(acc, w_ref[...]) + b_ref[...]
    hu = _dot(h, u_ref[...])
    out_ref[...] = _gru_update(h, xwb, hu)


def _link_body(qg_ref, ls_ref, qs_ref, w_ref, u_ref, b_ref, ls_out_ref, tab_ref):
    w = w_ref[...]
    u = u_ref[...]
    b = b_ref[...]
    h = ls_ref[...]
    for q in range(QL):
        x = qg_ref[pl.ds(q * L, L), :]
        xwb = _dot(x, w) + b
        hu = _dot(h, u)
        h = _gru_update(h, xwb, hu)
    ls_out_ref[...] = h
    tab_ref[:, :H] = qs_ref[...]
    tab_ref[:, H:] = h


def _init_body(tg_ref, cap_ref, bt_ref, lew1_ref, leb1_ref, lew2_ref, leb2_ref,
               qew1_ref, qeb1_ref, qew2_ref, qeb2_ref,
               ls_ref, qs_ref, tab_ref, icap_ref):
    acc = tg_ref[pl.ds(0, L), :]
    for j in range(1, PL):
        acc = acc + tg_ref[pl.ds(j * L, L), :]
    cap = cap_ref[...]
    load = acc[:, 0:1] / cap
    l1 = jnp.maximum(load * lew1_ref[0:1, :] + leb1_ref[...], 0.0)
    ls = jnp.maximum(_dot(l1, lew2_ref[...]) + leb2_ref[...], 0.0)
    bt = bt_ref[...]
    q1 = jnp.where(bt == 0, qew1_ref[0:1, :], qew1_ref[1:2, :]) + qeb1_ref[...]
    q1 = jnp.maximum(q1, 0.0)
    qs = jnp.maximum(_dot(q1, qew2_ref[...]) + qeb2_ref[...], 0.0)
    ls_ref[...] = ls
    qs_ref[...] = qs
    tab_ref[:, :H] = qs
    tab_ref[:, H:] = ls
    icap_ref[...] = jnp.broadcast_to(1.0 / (cap * 1e9), (L, 16))


def _readout_body(pss_ref, icap_ref, len_ref, tr_ref, pk_ref,
                  w1_ref, b1_ref, w2_ref, b2_ref, w3_ref, b3_ref, out_ref):
    w1 = w1_ref[...]
    b1 = b1_ref[...]
    w2 = w2_ref[...]
    b2 = b2_ref[...]
    w3 = w3_ref[...]
    b3 = b3_ref[...]
    length = len_ref[...]
    qd = jnp.zeros((BLK, 1), jnp.float32)
    td = jnp.zeros((BLK, 1), jnp.float32)
    for p in range(P):
        hp = pss_ref[p + 1]
        h1 = jnp.maximum(_dot(hp, w1) + b1, 0.0)
        h2 = jnp.maximum(_dot(h1, w2) + b2, 0.0)
        occ = _dot(h2, w3) + b3
        ic = icap_ref[p][:, 0:1]
        m = (length > p).astype(jnp.float32)
        qd = qd + occ * ic * m
        td = td + ic * m
    out_ref[...] = qd + (tr_ref[...] / pk_ref[...]) * td


_GRID = FP // BLK


def _full(shape):
    return pl.BlockSpec(shape, lambda i: tuple(0 for _ in shape))


def _emb_call(fps, ipg, w, u, b):
    return pl.pallas_call(
        _emb_body,
        grid=(_GRID,),
        in_specs=[
            pl.BlockSpec((BLK, T), lambda i: (i, 0)),
            pl.BlockSpec((BLK, T), lambda i: (i, 0)),
            _full((2, 3 * H)),
            _full((H, 3 * H)),
            _full((1, 3 * H)),
        ],
        out_specs=pl.BlockSpec((BLK, H), lambda i: (i, 0)),
        out_shape=jax.ShapeDtypeStruct((FP, H), jnp.float32),
    )(fps, ipg, w, u, b)


def _path_call(xcat, h0, w, u, b):
    return pl.pallas_call(
        _path_body,
        grid=(_GRID,),
        in_specs=[
            pl.BlockSpec((P, BLK, 2 * H), lambda i: (0, i, 0)),
            pl.BlockSpec((BLK, H), lambda i: (i, 0)),
            _full((2 * H, 3 * H)),
            _full((H, 3 * H)),
            _full((1, 3 * H)),
        ],
        out_specs=[
            pl.BlockSpec((P + 1, BLK, H), lambda i: (0, i, 0)),
            pl.BlockSpec((BLK, H), lambda i: (i, 0)),
        ],
        out_shape=[
            jax.ShapeDtypeStruct((P + 1, FP, H), jnp.float32),
            jax.ShapeDtypeStruct((FP, H), jnp.float32),
        ],
    )(xcat, h0, w, u, b)


def _queue_call(gb, qs, w, u, b):
    return pl.pallas_call(
        _queue_body,
        out_shape=jax.ShapeDtypeStruct((L, H), jnp.float32),
    )(gb, qs, w, u, b)


def _link_call(qg, ls, qs, w, u, b):
    return pl.pallas_call(
        _link_body,
        out_shape=[
            jax.ShapeDtypeStruct((L, H), jnp.float32),
            jax.ShapeDtypeStruct((L, 2 * H), jnp.float32),
        ],
    )(qg, ls, qs, w, u, b)


def _init_call(tg, cap, bt, lew1, leb1, lew2, leb2, qew1, qeb1, qew2, qeb2):
    return pl.pallas_call(
        _init_body,
        out_shape=[
            jax.ShapeDtypeStruct((L, H), jnp.float32),
            jax.ShapeDtypeStruct((L, H), jnp.float32),
            jax.ShapeDtypeStruct((L, 2 * H), jnp.float32),
            jax.ShapeDtypeStruct((L, 16), jnp.float32),
        ],
    )(tg, cap, bt, lew1, leb1, lew2, leb2, qew1, qeb1, qew2, qeb2)


def _readout_call(pss, icap_g, length, tr, pk, w1, b1, w2, b2, w3, b3):
    return pl.pallas_call(
        _readout_body,
        grid=(_GRID,),
        in_specs=[
            pl.BlockSpec((P + 1, BLK, H), lambda i: (0, i, 0)),
            pl.BlockSpec((P, BLK, 16), lambda i: (0, i, 0)),
            pl.BlockSpec((BLK, 1), lambda i: (i, 0)),
            pl.BlockSpec((BLK, 1), lambda i: (i, 0)),
            pl.BlockSpec((BLK, 1), lambda i: (i, 0)),
            _full((H, H // 2)),
            _full((1, H // 2)),
            _full((H // 2, H // 2)),
            _full((1, H // 2)),
            _full((H // 2, 1)),
            _full((1, 1)),
        ],
        out_specs=pl.BlockSpec((BLK, 1), lambda i: (i, 0)),
        out_shape=jax.ShapeDtypeStruct((FP, 1), jnp.float32),
    )(pss, icap_g, length, tr, pk, w1, b1, w2, b2, w3, b3)


def kernel(flow_traffic, flow_packets, flow_length, link_capacity, buffer_type,
           link_to_path, path_to_link_path, path_to_link_pos, queue_to_link,
           flow_packet_size_wt_0, flow_ipg_wt_0,
           emb_W, emb_U, emb_b, pu_W, pu_U, pu_b, qu_W, qu_U, qu_b,
           lu_W, lu_U, lu_b, qe_W1, qe_b1, qe_W2, qe_b2, le_W1, le_b1,
           le_W2, le_b2, ro_W1, ro_b1, ro_W2, ro_b2, ro_W3, ro_b3):
    f32 = jnp.float32
    i32 = jnp.int32

    ltp = link_to_path.astype(i32)
    idx_a = jnp.pad(ltp.T, ((0, 0), (0, FP - F))).reshape(-1)
    flat_b = (path_to_link_pos.astype(i32) * FP + path_to_link_path.astype(i32))
    idx_b = jnp.pad(flat_b.T.reshape(-1), (0, _NB - L * PL))
    idx_c = jnp.pad(queue_to_link.astype(i32).T.reshape(-1), (0, _NC - L * QL))
    idx_g1 = jnp.pad(path_to_link_path.astype(i32).T.reshape(-1), (0, _NB - L * PL))

    fps = jnp.pad(flow_packet_size_wt_0.astype(f32).reshape(F, T), ((0, FP - F), (0, 0)))
    ipg = jnp.pad(flow_ipg_wt_0.astype(f32).reshape(F, T), ((0, FP - F), (0, 0)))
    length_p = jnp.pad(flow_length.astype(i32), ((0, FP - F), (0, 0)))
    tr_p = jnp.pad(flow_traffic.astype(f32), ((0, FP - F), (0, 0)))
    pk_p = jnp.pad(flow_packets.astype(f32), ((0, FP - F), (0, 0)),
                   constant_values=1.0)
    traffic16 = jnp.broadcast_to(flow_traffic.astype(f32), (F, 16))

    emb_b2d = emb_b.reshape(1, 3 * H)
    pu_b2d = pu_b.reshape(1, 3 * H)
    qu_b2d = qu_b.reshape(1, 3 * H)
    lu_b2d = lu_b.reshape(1, 3 * H)

    tg = _sc_gather(traffic16, idx_g1, 16)
    ls, qs, table, icap16 = _init_call(
        tg, link_capacity.astype(f32), buffer_type.astype(i32),
        le_W1, le_b1.reshape(1, H), le_W2, le_b2.reshape(1, H),
        qe_W1, qe_b1.reshape(1, H), qe_W2, qe_b2.reshape(1, H))
    icap_g = _sc_gather(icap16, idx_a, 16).reshape(P, FP, 16)
    h = _emb_call(fps, ipg, emb_W, emb_U, emb_b2d)

    pss = None
    for _ in range(ITERS):
        xcat = _sc_gather(table, idx_a, 2 * H).reshape(P, FP, 2 * H)
        pss, h = _path_call(xcat, h, pu_W, pu_U, pu_b2d)
        gb = _sc_gather(pss.reshape((P + 1) * FP, H), idx_b, H)
        qs = _queue_call(gb, qs, qu_W, qu_U, qu_b2d)
        qg = _sc_gather(qs, idx_c, H)
        ls, table = _link_call(qg, ls, qs, lu_W, lu_U, lu_b2d)

    out = _readout_call(
        pss, icap_g, length_p, tr_p, pk_p,
        ro_W1, ro_b1.reshape(1, H // 2), ro_W2, ro_b2.reshape(1, H // 2),
        ro_W3, ro_b3.reshape(1, 1))
    return out[:F]


# speedup vs baseline: 2.0742x; 1.0344x over previous
import functools

import jax
import jax.numpy as jnp
from jax.experimental import pallas as pl
from jax.experimental.pallas import tpu as pltpu
from jax.experimental.pallas import tpu_sc as plsc

H = 32
ITERS = 8
F = 10000
FP = 10240
L = 2000
P = 8
PL = 32
QL = 4
T = 16
BLK = 1024

_GW = 128
_NA = P * FP
_NB = 65536
_NC = 8192


_NW = 32


def _sc_gather(table, idx_flat, width):
    n = idx_flat.shape[0]
    r = n // _NW
    dt = table.dtype
    ch = r
    while ch * width * 4 > 200 * 1024:
        ch //= 2
    nch = r // ch
    mesh = plsc.VectorSubcoreMesh(core_axis_name="core", subcore_axis_name="subcore")
    scratch = [
        pltpu.VMEM((r,), jnp.int32),
        pltpu.VMEM((ch, width), dt),
        pltpu.VMEM((ch, width), dt),
        pltpu.SemaphoreType.DMA,
        pltpu.SemaphoreType.DMA,
        pltpu.SemaphoreType.DMA,
        pltpu.SemaphoreType.DMA,
    ]

    @functools.partial(
        pl.kernel,
        out_type=jax.ShapeDtypeStruct((n, width), dt),
        mesh=mesh,
        scratch_types=scratch,
        compiler_params=pltpu.CompilerParams(use_tc_tiling_on_sc=False),
    )
    def gk(tab_hbm, idx_hbm, out_hbm, idx_v, buf0, buf1, gs0, gs1, os0, os1):
        wid = jax.lax.axis_index("subcore") * 2 + jax.lax.axis_index("core")
        base = wid * r
        pltpu.sync_copy(idx_hbm.at[pl.ds(base, r)], idx_v)
        bufs = (buf0, buf1)
        gsems = (gs0, gs1)
        osems = (os0, os1)
        gh = [None] * nch
        oh = [None] * nch

        def fire_g(c):
            gh[c] = pltpu.async_copy(
                tab_hbm.at[idx_v.at[pl.ds(c * ch, ch)]], bufs[c & 1], gsems[c & 1])

        def fire_o(c):
            oh[c] = pltpu.async_copy(
                bufs[c & 1], out_hbm.at[pl.ds(base + c * ch, ch)], osems[c & 1])

        fire_g(0)
        if nch > 1:
            fire_g(1)
        for c in range(nch):
            gh[c].wait()
            fire_o(c)
            if c + 2 < nch:
                oh[c].wait()
                fire_g(c + 2)
        for c in range(max(0, nch - 2), nch):
            oh[c].wait()

    return gk(table, idx_flat)


def _gru_update(h, xwb, hu):
    z = jax.nn.sigmoid(xwb[:, :H] + hu[:, :H])
    r = jax.nn.sigmoid(xwb[:, H:2 * H] + hu[:, H:2 * H])
    n = jnp.tanh(xwb[:, 2 * H:] + r * hu[:, 2 * H:])
    return z * h + (1.0 - z) * n


def _dot(a, b):
    return jnp.dot(a, b, preferred_element_type=jnp.float32)


def _emb_body(fps_ref, ipg_ref, w_ref, u_ref, b_ref, out_ref):
    b = b_ref[...]
    w0 = w_ref[0:1, :]
    w1 = w_ref[1:2, :]
    u = u_ref[...]
    h = jnp.zeros((BLK, H), jnp.float32)
    for t in range(T):
        xwb = fps_ref[:, t:t + 1] * w0 + ipg_ref[:, t:t + 1] * w1 + b
        hu = _dot(h, u)
        h = _gru_update(h, xwb, hu)
    out_ref[...] = h


def _path_body(xcat_ref, h0_ref, w_ref, u_ref, b_ref, pss_ref, ht_ref):
    w = w_ref[...]
    u = u_ref[...]
    b = b_ref[...]
    h = h0_ref[...]
    pss_ref[0] = h
    for p in range(P):
        x = xcat_ref[p]
        xwb = _dot(x, w) + b
        hu = _dot(h, u)
        h = _gru_update(h, xwb, hu)
        pss_ref[p + 1] = h
    ht_ref[...] = h


def _queue_body(gb_ref, qs_ref, w_ref, u_ref, b_ref, out_ref):
    acc = gb_ref[pl.ds(0, L), :]
    for j in range(1, PL):
        acc = acc + gb_ref[pl.ds(j * L, L), :]
    h = qs_ref[...]
    xwb = _dot(acc, w_ref[...]) + b_ref[...]
    hu = _dot(h, u_ref[...])
    out_ref[...] = _gru_update(h, xwb, hu)


def _link_body(qg_ref, ls_ref, qs_ref, w_ref, u_ref, b_ref, ls_out_ref, tab_ref):
    w = w_ref[...]
    u = u_ref[...]
    b = b_ref[...]
    h = ls_ref[...]
    for q in range(QL):
        x = qg_ref[pl.ds(q * L, L), :]
        xwb = _dot(x, w) + b
        hu = _dot(h, u)
        h = _gru_update(h, xwb, hu)
    ls_out_ref[...] = h
    tab_ref[:, :H] = qs_ref[...]
    tab_ref[:, H:] = h


def _init_body(tg_ref, cap_ref, bt_ref, lew1_ref, leb1_ref, lew2_ref, leb2_ref,
               qew1_ref, qeb1_ref, qew2_ref, qeb2_ref,
               ls_ref, qs_ref, tab_ref, icap_ref):
    acc = tg_ref[pl.ds(0, L), :]
    for j in range(1, PL):
        acc = acc + tg_ref[pl.ds(j * L, L), :]
    cap = cap_ref[...]
    load = acc[:, 0:1] / cap
    l1 = jnp.maximum(load * lew1_ref[0:1, :] + leb1_ref[...], 0.0)
    ls = jnp.maximum(_dot(l1, lew2_ref[...]) + leb2_ref[...], 0.0)
    bt = bt_ref[...]
    q1 = jnp.where(bt == 0, qew1_ref[0:1, :], qew1_ref[1:2, :]) + qeb1_ref[...]
    q1 = jnp.maximum(q1, 0.0)
    qs = jnp.maximum(_dot(q1, qew2_ref[...]) + qeb2_ref[...], 0.0)
    ls_ref[...] = ls
    qs_ref[...] = qs
    tab_ref[:, :H] = qs
    tab_ref[:, H:] = ls
    icap_ref[...] = jnp.broadcast_to(1.0 / (cap * 1e9), (L, 16))


def _readout_body(pss_ref, icap_ref, len_ref, tr_ref, pk_ref,
                  w1_ref, b1_ref, w2_ref, b2_ref, w3_ref, b3_ref, out_ref):
    w1 = w1_ref[...]
    b1 = b1_ref[...]
    w2 = w2_ref[...]
    b2 = b2_ref[...]
    w3 = w3_ref[...]
    b3 = b3_ref[...]
    length = len_ref[...]
    qd = jnp.zeros((BLK, 1), jnp.float32)
    td = jnp.zeros((BLK, 1), jnp.float32)
    for p in range(P):
        hp = pss_ref[p + 1]
        h1 = jnp.maximum(_dot(hp, w1) + b1, 0.0)
        h2 = jnp.maximum(_dot(h1, w2) + b2, 0.0)
        occ = _dot(h2, w3) + b3
        ic = icap_ref[p][:, 0:1]
        m = (length > p).astype(jnp.float32)
        qd = qd + occ * ic * m
        td = td + ic * m
    out_ref[...] = qd + (tr_ref[...] / pk_ref[...]) * td


_GRID = FP // BLK


def _full(shape):
    return pl.BlockSpec(shape, lambda i: tuple(0 for _ in shape))


def _emb_call(fps, ipg, w, u, b):
    return pl.pallas_call(
        _emb_body,
        grid=(_GRID,),
        in_specs=[
            pl.BlockSpec((BLK, T), lambda i: (i, 0)),
            pl.BlockSpec((BLK, T), lambda i: (i, 0)),
            _full((2, 3 * H)),
            _full((H, 3 * H)),
            _full((1, 3 * H)),
        ],
        out_specs=pl.BlockSpec((BLK, H), lambda i: (i, 0)),
        out_shape=jax.ShapeDtypeStruct((FP, H), jnp.float32),
    )(fps, ipg, w, u, b)


def _path_call(xcat, h0, w, u, b):
    return pl.pallas_call(
        _path_body,
        grid=(_GRID,),
        in_specs=[
            pl.BlockSpec((P, BLK, 2 * H), lambda i: (0, i, 0)),
            pl.BlockSpec((BLK, H), lambda i: (i, 0)),
            _full((2 * H, 3 * H)),
            _full((H, 3 * H)),
            _full((1, 3 * H)),
        ],
        out_specs=[
            pl.BlockSpec((P + 1, BLK, H), lambda i: (0, i, 0)),
            pl.BlockSpec((BLK, H), lambda i: (i, 0)),
        ],
        out_shape=[
            jax.ShapeDtypeStruct((P + 1, FP, H), jnp.float32),
            jax.ShapeDtypeStruct((FP, H), jnp.float32),
        ],
    )(xcat, h0, w, u, b)


def _queue_call(gb, qs, w, u, b):
    return pl.pallas_call(
        _queue_body,
        out_shape=jax.ShapeDtypeStruct((L, H), jnp.float32),
    )(gb, qs, w, u, b)


def _link_call(qg, ls, qs, w, u, b):
    return pl.pallas_call(
        _link_body,
        out_shape=[
            jax.ShapeDtypeStruct((L, H), jnp.float32),
            jax.ShapeDtypeStruct((L, 2 * H), jnp.float32),
        ],
    )(qg, ls, qs, w, u, b)


def _init_call(tg, cap, bt, lew1, leb1, lew2, leb2, qew1, qeb1, qew2, qeb2):
    return pl.pallas_call(
        _init_body,
        out_shape=[
            jax.ShapeDtypeStruct((L, H), jnp.float32),
            jax.ShapeDtypeStruct((L, H), jnp.float32),
            jax.ShapeDtypeStruct((L, 2 * H), jnp.float32),
            jax.ShapeDtypeStruct((L, 16), jnp.float32),
        ],
    )(tg, cap, bt, lew1, leb1, lew2, leb2, qew1, qeb1, qew2, qeb2)


def _readout_call(pss, icap_g, length, tr, pk, w1, b1, w2, b2, w3, b3):
    return pl.pallas_call(
        _readout_body,
        grid=(_GRID,),
        in_specs=[
            pl.BlockSpec((P + 1, BLK, H), lambda i: (0, i, 0)),
            pl.BlockSpec((P, BLK, 16), lambda i: (0, i, 0)),
            pl.BlockSpec((BLK, 1), lambda i: (i, 0)),
            pl.BlockSpec((BLK, 1), lambda i: (i, 0)),
            pl.BlockSpec((BLK, 1), lambda i: (i, 0)),
            _full((H, H // 2)),
            _full((1, H // 2)),
            _full((H // 2, H // 2)),
            _full((1, H // 2)),
            _full((H // 2, 1)),
            _full((1, 1)),
        ],
        out_specs=pl.BlockSpec((BLK, 1), lambda i: (i, 0)),
        out_shape=jax.ShapeDtypeStruct((FP, 1), jnp.float32),
    )(pss, icap_g, length, tr, pk, w1, b1, w2, b2, w3, b3)


def kernel(flow_traffic, flow_packets, flow_length, link_capacity, buffer_type,
           link_to_path, path_to_link_path, path_to_link_pos, queue_to_link,
           flow_packet_size_wt_0, flow_ipg_wt_0,
           emb_W, emb_U, emb_b, pu_W, pu_U, pu_b, qu_W, qu_U, qu_b,
           lu_W, lu_U, lu_b, qe_W1, qe_b1, qe_W2, qe_b2, le_W1, le_b1,
           le_W2, le_b2, ro_W1, ro_b1, ro_W2, ro_b2, ro_W3, ro_b3):
    f32 = jnp.float32
    i32 = jnp.int32

    ltp = link_to_path.astype(i32)
    idx_a = jnp.pad(ltp.T, ((0, 0), (0, FP - F))).reshape(-1)
    flat_b = (path_to_link_pos.astype(i32) * FP + path_to_link_path.astype(i32))
    idx_b = jnp.pad(flat_b.T.reshape(-1), (0, _NB - L * PL))
    idx_c = jnp.pad(queue_to_link.astype(i32).T.reshape(-1), (0, _NC - L * QL))
    idx_g1 = jnp.pad(path_to_link_path.astype(i32).T.reshape(-1), (0, _NB - L * PL))

    fps = jnp.pad(flow_packet_size_wt_0.astype(f32).reshape(F, T), ((0, FP - F), (0, 0)))
    ipg = jnp.pad(flow_ipg_wt_0.astype(f32).reshape(F, T), ((0, FP - F), (0, 0)))
    length_p = jnp.pad(flow_length.astype(i32), ((0, FP - F), (0, 0)))
    tr_p = jnp.pad(flow_traffic.astype(f32), ((0, FP - F), (0, 0)))
    pk_p = jnp.pad(flow_packets.astype(f32), ((0, FP - F), (0, 0)),
                   constant_values=1.0)
    traffic16 = jnp.broadcast_to(flow_traffic.astype(f32), (F, 16))

    emb_b2d = emb_b.reshape(1, 3 * H)
    pu_b2d = pu_b.reshape(1, 3 * H)
    qu_b2d = qu_b.reshape(1, 3 * H)
    lu_b2d = lu_b.reshape(1, 3 * H)

    tg = _sc_gather(traffic16, idx_g1, 16)
    ls, qs, table, icap16 = _init_call(
        tg, link_capacity.astype(f32), buffer_type.astype(i32),
        le_W1, le_b1.reshape(1, H), le_W2, le_b2.reshape(1, H),
        qe_W1, qe_b1.reshape(1, H), qe_W2, qe_b2.reshape(1, H))
    icap_g = _sc_gather(icap16, idx_a, 16).reshape(P, FP, 16)
    h = _emb_call(fps, ipg, emb_W, emb_U, emb_b2d)

    pss = None
    for _ in range(ITERS):
        xcat = _sc_gather(table, idx_a, 2 * H).reshape(P, FP, 2 * H)
        pss, h = _path_call(xcat, h, pu_W, pu_U, pu_b2d)
        gb = _sc_gather(pss.reshape((P + 1) * FP, H), idx_b, H)
        qs = _queue_call(gb, qs, qu_W, qu_U, qu_b2d)
        qg = _sc_gather(qs, idx_c, H)
        ls, table = _link_call(qg, ls, qs, lu_W, lu_U, lu_b2d)

    out = _readout_call(
        pss, icap_g, length_p, tr_p, pk_p,
        ro_W1, ro_b1.reshape(1, H // 2), ro_W2, ro_b2.reshape(1, H // 2),
        ro_W3, ro_b3.reshape(1, 1))
    return out[:F]
